# baseline (speedup 1.0000x reference)
import functools

import jax
import jax.numpy as jnp
from jax import lax
from jax.experimental import pallas as pl
from jax.experimental.pallas import tpu as pltpu
from jax.experimental.pallas import tpu_sc as plsc

_BATCH = 4096
_SEQ = 200
_D = 64
_TOTAL = _BATCH * _SEQ

_NC = 2
_NS = 16
_NW = _NC * _NS
_PER_W = _TOTAL // _NW

_CHUNK = 128
_BANK_CHUNKS = 5
_BANK_ROWS = _BANK_CHUNKS * _CHUNK
_N_CYCLES = _PER_W // _BANK_ROWS
_N_LAPS = _N_CYCLES // 2


def _emb_body(idx_hbm, table_hbm, out_hbm, idx_v, rows_v, gsem, ssem0, ssem1):
    ssems = (ssem0, ssem1)
    wid = lax.axis_index("s") * _NC + lax.axis_index("c")
    base = wid * _PER_W

    pltpu.sync_copy(idx_hbm.at[pl.ds(base, _PER_W)], idx_v)

    def fire_gathers(cycle, bank):
        for j in range(_BANK_CHUNKS):
            idx_slice = idx_v.at[pl.ds(cycle * _BANK_ROWS + j * _CHUNK, _CHUNK)]
            pltpu.async_copy(
                table_hbm.at[idx_slice],
                rows_v.at[bank].at[pl.ds(j * _CHUNK, _CHUNK)],
                gsem,
            )

    def drain_gathers(bank):
        pltpu.make_async_copy(
            table_hbm.at[pl.ds(0, _BANK_ROWS)], rows_v.at[bank], gsem
        ).wait()

    def start_store(cycle, bank):
        pltpu.async_copy(
            rows_v.at[bank],
            out_hbm.at[pl.ds(base + cycle * _BANK_ROWS, _BANK_ROWS)],
            ssems[bank],
        )

    def wait_store(bank):
        pltpu.make_async_copy(
            rows_v.at[bank], out_hbm.at[pl.ds(base, _BANK_ROWS)], ssems[bank]
        ).wait()

    for bank in (0, 1):
        fire_gathers(bank, bank)
        drain_gathers(bank)
        start_store(bank, bank)

    def lap(t, carry):
        for bank in (0, 1):
            cycle = 2 * t + bank
            wait_store(bank)
            fire_gathers(cycle, bank)
            drain_gathers(bank)
            start_store(cycle, bank)
        return carry

    lax.fori_loop(1, _N_LAPS, lap, 0)

    wait_store(0)
    wait_store(1)


_mesh = plsc.VectorSubcoreMesh(core_axis_name="c", subcore_axis_name="s")

_emb = functools.partial(
    pl.kernel,
    out_type=jax.ShapeDtypeStruct((_TOTAL, _D), jnp.float32),
    mesh=_mesh,
    scratch_types=[
        pltpu.VMEM((_PER_W,), jnp.int32),
        pltpu.VMEM((2, _BANK_ROWS, _D), jnp.float32),
        pltpu.SemaphoreType.DMA,
        pltpu.SemaphoreType.DMA,
        pltpu.SemaphoreType.DMA,
    ],
    compiler_params=pltpu.CompilerParams(use_tc_tiling_on_sc=False),
)(_emb_body)


@jax.jit
def kernel(inputs, table):
    flat_idx = inputs.reshape(-1).astype(jnp.int32)
    out = _emb(flat_idx, table)
    return out.reshape(_BATCH, _SEQ, _D)
